# baseline (speedup 1.0000x reference)
import functools

import jax
import jax.numpy as jnp
from jax import lax
from jax.experimental import pallas as pl
from jax.experimental.pallas import tpu as pltpu

K = 32
CP = 48
QT = 1024
NCHUNK = 512


def _preconv_body(x_ref, w0_ref, b0_ref, w1_ref, b1_ref, h_ref, ft_ref):
    x = x_ref[0]
    f0 = jnp.maximum(jnp.dot(w0_ref[...], x, preferred_element_type=jnp.float32)
                     + b0_ref[...], 0.0)
    f = jnp.maximum(jnp.dot(w1_ref[...], f0, preferred_element_type=jnp.float32)
                    + b1_ref[...], 0.0)
    h_ref[0] = f
    ft = jnp.transpose(f)
    ft_ref[0, :, 0:CP] = ft
    ft_ref[0, :, CP:] = jnp.zeros_like(ft_ref[0, :, CP:])


def _assa_body(s_ref, st_ref, qt_ref, h_ref, fq_ref, w2_ref, b2_ref,
               wskip_ref, out_ref, d_ref):
    S = s_ref[0]
    q = qt_ref[0]
    N = S.shape[0]
    ss = jnp.sum(S * S, axis=1, keepdims=True)
    qq = jnp.sum(q * q, axis=0, keepdims=True)
    d_ref[...] = (ss + qq
                  - 2.0 * jnp.dot(S, q, preferred_element_type=jnp.float32))

    nchunks = N // NCHUNK

    def g_of(x):
        b = lax.bitcast_convert_type(x, jnp.int32)
        return jnp.where(b >= 0, b, b ^ jnp.int32(0x7FFFFFFF))

    def ginv(gbits):
        b = jnp.where(gbits >= 0, gbits, gbits ^ jnp.int32(0x7FFFFFFF))
        return lax.bitcast_convert_type(b, jnp.float32)

    def cseed(c, carry):
        mn, bd, sm = carry
        base = pl.multiple_of(c * 128, 128)
        blkmin = jnp.min(d_ref[pl.ds(base, 128), :], axis=0, keepdims=True)
        return (jnp.minimum(mn, blkmin), jnp.maximum(bd, blkmin), sm + blkmin)
    mn0, bd0, sum0 = lax.fori_loop(
        0, N // 128, cseed,
        (jnp.full((1, QT), jnp.inf, jnp.float32),
         jnp.full((1, QT), -jnp.inf, jnp.float32),
         jnp.zeros((1, QT), jnp.float32)))
    tmean = sum0 * (1.0 / (N // 128))

    lo0 = g_of(mn0) - 1
    hi0 = g_of(bd0)
    state0 = (lo0, hi0,
              jnp.zeros((1, QT), jnp.float32),
              jnp.full((1, QT), 3.0 * K, jnp.float32),
              ginv(hi0),
              jnp.zeros((1, QT), jnp.float32),
              jnp.int32(0))

    def q_cond(st):
        _, _, _, _, _, done, it = st
        return jnp.logical_and(jnp.any(done < 0.5), it < 64)

    def q_body(st):
        lo, hi, clo, chi, tstar, done, it = st
        t_lo, t_hi = ginv(lo), ginv(hi)
        denom = jnp.maximum(chi - clo, 1.0)
        tcand = t_lo + (t_hi - t_lo) * ((float(K) + 0.5) - clo) / denom
        pcand = jnp.where(it == 0, g_of(tmean), g_of(tcand))
        pbis = lo + lax.shift_right_logical(hi - lo, 1)
        pa = jnp.clip(pcand, lo + 1, jnp.maximum(lo + 1, hi - 1))
        pb = jnp.clip(pbis, lo + 1, jnp.maximum(lo + 1, hi - 1))
        p1 = jnp.minimum(pa, pb)
        p2 = jnp.maximum(pa, pb)
        t1, t2 = ginv(p1), ginv(p2)
        dd = d_ref[...]
        c1 = jnp.sum(jnp.where(dd <= t1, 1.0, 0.0), axis=0, keepdims=True)
        c2 = jnp.sum(jnp.where(dd <= t2, 1.0, 0.0), axis=0, keepdims=True)
        notdone = done < 0.5
        hit1 = c1 == float(K)
        hit2 = jnp.logical_and(c2 == float(K), jnp.logical_not(hit1))
        hit = jnp.logical_and(jnp.logical_or(hit1, hit2), notdone)
        gapdone = jnp.logical_and((hi - lo) <= 1,
                                  jnp.logical_and(notdone, jnp.logical_not(hit)))
        tstar = jnp.where(jnp.logical_and(notdone, hit1), t1,
                jnp.where(jnp.logical_and(notdone, hit2), t2,
                jnp.where(gapdone, t_hi, tstar)))
        done = jnp.maximum(done, jnp.where(
            jnp.logical_or(hit, gapdone), 1.0, 0.0))
        upd = done < 0.5
        b2lt = c2 < float(K)
        b1lt = c1 < float(K)
        lo = jnp.where(jnp.logical_and(upd, b2lt), p2,
             jnp.where(jnp.logical_and(upd, b1lt), p1, lo))
        clo = jnp.where(jnp.logical_and(upd, b2lt), c2,
              jnp.where(jnp.logical_and(upd, b1lt), c1, clo))
        g1 = jnp.logical_not(b1lt)
        g2 = jnp.logical_not(b2lt)
        hi = jnp.where(jnp.logical_and(upd, g1), p1,
             jnp.where(jnp.logical_and(upd, g2), p2, hi))
        chi = jnp.where(jnp.logical_and(upd, g1), c1,
              jnp.where(jnp.logical_and(upd, g2), c2, chi))
        return lo, hi, clo, chi, tstar, done, it + 1

    _, hi_f, _, _, tstar_f, done_f, _ = lax.while_loop(q_cond, q_body, state0)
    t_v = jnp.where(done_f > 0.5, tstar_f, ginv(hi_f))
    MT = (d_ref[...] <= t_v).astype(jnp.bfloat16)
    f = h_ref[0]
    st = st_ref[0]
    H = jnp.concatenate(
        [f, f * st[0:1], f * st[1:2], f * st[2:3]],
        axis=0).astype(jnp.bfloat16)
    ST = jnp.dot(H, MT, preferred_element_type=jnp.float32) * (1.0 / K)

    fq = fq_ref[0, :, 0:CP]

    G = ST[0:CP]
    A = jnp.concatenate([
        ST[CP:2 * CP] - q[0:1] * G,
        ST[2 * CP:3 * CP] - q[1:2] * G,
        ST[3 * CP:4 * CP] - q[2:3] * G,
    ], axis=0)
    term = jnp.dot(w2_ref[...], A, preferred_element_type=jnp.float32) + b2_ref[...]
    skip = lax.dot_general(wskip_ref[...], fq, (((1,), (1,)), ((), ())),
                           preferred_element_type=jnp.float32)
    out_ref[0] = jnp.maximum(term + skip, 0.0)


def _sc_gather_rows(table, idx):
    from jax.experimental.pallas import tpu_sc as plsc
    info = plsc.get_sparse_core_info()
    nw = info.num_cores * info.num_subcores
    bq = idx.shape[0]
    bpw = bq // nw
    mesh = plsc.VectorSubcoreMesh(core_axis_name="c", subcore_axis_name="s")

    @functools.partial(
        pl.kernel, mesh=mesh,
        out_type=jax.ShapeDtypeStruct((bq, table.shape[1]), jnp.float32),
        scratch_types=[
            pltpu.VMEM((bpw,), jnp.int32),
            pltpu.VMEM((bpw, table.shape[1]), jnp.float32),
            pltpu.SemaphoreType.DMA,
        ],
    )
    def gk(tab_hbm, idx_hbm, out_hbm, idx_v, rows_v, sem):
        wid = lax.axis_index("s") * info.num_cores + lax.axis_index("c")
        base = wid * bpw
        pltpu.sync_copy(idx_hbm.at[pl.ds(base, bpw)], idx_v)
        pltpu.async_copy(tab_hbm.at[idx_v], rows_v, sem).wait()
        pltpu.sync_copy(rows_v, out_hbm.at[pl.ds(base, bpw)])

    return gk(table, idx)


def kernel(query_xyz, support_xyz, features, query_idx, W0, b0, W1, b1, W2, b2, Wskip):
    B, NP, _ = query_xyz.shape
    N = support_xyz.shape[1]
    Cin = features.shape[1]
    Cmid = W1.shape[0]
    Cout = W2.shape[0]

    W1p = jnp.pad(W1, ((0, CP - Cmid), (0, 0)))
    b1p = jnp.pad(b1, (0, CP - Cmid))[:, None]
    W2p = jnp.pad(W2.reshape(Cout, 3, Cmid), ((0, 0), (0, 0), (0, CP - Cmid))
                  ).reshape(Cout, 3 * CP)
    Wskipp = jnp.pad(Wskip, ((0, 0), (0, CP - Cmid)))
    b0c = b0[:, None]
    b2c = b2[:, None]
    sT = jnp.transpose(support_xyz, (0, 2, 1))
    qT = jnp.transpose(query_xyz, (0, 2, 1))

    NT = 512
    H, fT = pl.pallas_call(
        _preconv_body,
        grid=(B, N // NT),
        in_specs=[
            pl.BlockSpec((1, Cin, NT), lambda b, n: (b, 0, n)),
            pl.BlockSpec((Cin, Cin), lambda b, n: (0, 0)),
            pl.BlockSpec((Cin, 1), lambda b, n: (0, 0)),
            pl.BlockSpec((CP, Cin), lambda b, n: (0, 0)),
            pl.BlockSpec((CP, 1), lambda b, n: (0, 0)),
        ],
        out_specs=[
            pl.BlockSpec((1, CP, NT), lambda b, n: (b, 0, n)),
            pl.BlockSpec((1, NT, 128), lambda b, n: (b, n, 0)),
        ],
        out_shape=[
            jax.ShapeDtypeStruct((B, CP, N), jnp.float32),
            jax.ShapeDtypeStruct((B, N, 128), jnp.float32),
        ],
        compiler_params=pltpu.CompilerParams(
            dimension_semantics=("parallel", "parallel")),
    )(features, W0, b0c, W1p, b1p)

    flat_idx = (query_idx.astype(jnp.int32)
                + jnp.arange(B, dtype=jnp.int32)[:, None] * N).reshape(-1)
    fq = _sc_gather_rows(fT.reshape(B * N, 128), flat_idx).reshape(B, NP, 128)

    out = pl.pallas_call(
        _assa_body,
        grid=(B, NP // QT),
        in_specs=[
            pl.BlockSpec((1, N, 3), lambda b, t: (b, 0, 0)),
            pl.BlockSpec((1, 3, N), lambda b, t: (b, 0, 0)),
            pl.BlockSpec((1, 3, QT), lambda b, t: (b, 0, t)),
            pl.BlockSpec((1, CP, N), lambda b, t: (b, 0, 0)),
            pl.BlockSpec((1, QT, 128), lambda b, t: (b, t, 0)),
            pl.BlockSpec((Cout, 3 * CP), lambda b, t: (0, 0)),
            pl.BlockSpec((Cout, 1), lambda b, t: (0, 0)),
            pl.BlockSpec((Cout, CP), lambda b, t: (0, 0)),
        ],
        out_specs=pl.BlockSpec((1, Cout, QT), lambda b, t: (b, 0, t)),
        out_shape=jax.ShapeDtypeStruct((B, Cout, NP), jnp.float32),
        scratch_shapes=[pltpu.VMEM((N, QT), jnp.float32)],
        compiler_params=pltpu.CompilerParams(
            dimension_semantics=("parallel", "parallel")),
    )(support_xyz, sT, qT, H, fq, W2p, b2c, Wskipp)
    return out


# speedup vs baseline: 1.0281x; 1.0281x over previous
import functools

import jax
import jax.numpy as jnp
from jax import lax
from jax.experimental import pallas as pl
from jax.experimental.pallas import tpu as pltpu

K = 32
CP = 48
QT = 512


def _preconv_body(x_ref, w0_ref, b0_ref, w1_ref, b1_ref, h_ref, ft_ref):
    x = x_ref[0]
    f0 = jnp.maximum(jnp.dot(w0_ref[...], x, preferred_element_type=jnp.float32)
                     + b0_ref[...], 0.0)
    f = jnp.maximum(jnp.dot(w1_ref[...], f0, preferred_element_type=jnp.float32)
                    + b1_ref[...], 0.0)
    h_ref[0] = f
    ft = jnp.transpose(f)
    ft_ref[0, :, 0:CP] = ft
    ft_ref[0, :, CP:] = jnp.zeros_like(ft_ref[0, :, CP:])


def _assa_body(s_ref, st_ref, qt_ref, h_ref, fq_ref, w2_ref, b2_ref,
               wskip_ref, out_ref, d_ref):
    S = s_ref[0]
    q = qt_ref[0]
    N = S.shape[0]
    ss = jnp.sum(S * S, axis=1, keepdims=True)
    qq = jnp.sum(q * q, axis=0, keepdims=True)
    d_ref[...] = (ss + qq
                  - 2.0 * jnp.dot(S, q, preferred_element_type=jnp.float32))


    def g_of(x):
        b = lax.bitcast_convert_type(x, jnp.int32)
        return jnp.where(b >= 0, b, b ^ jnp.int32(0x7FFFFFFF))

    def ginv(gbits):
        b = jnp.where(gbits >= 0, gbits, gbits ^ jnp.int32(0x7FFFFFFF))
        return lax.bitcast_convert_type(b, jnp.float32)

    def cseed(c, carry):
        mn, bd, sm = carry
        base = pl.multiple_of(c * 128, 128)
        blkmin = jnp.min(d_ref[pl.ds(base, 128), :], axis=0, keepdims=True)
        return (jnp.minimum(mn, blkmin), jnp.maximum(bd, blkmin), sm + blkmin)
    mn0, bd0, sum0 = lax.fori_loop(
        0, N // 128, cseed,
        (jnp.full((1, QT), jnp.inf, jnp.float32),
         jnp.full((1, QT), -jnp.inf, jnp.float32),
         jnp.zeros((1, QT), jnp.float32)))
    tmean = sum0 * (1.0 / (N // 128))

    lo0 = g_of(mn0) - 1
    hi0 = g_of(bd0)
    state0 = (lo0, hi0,
              jnp.zeros((1, QT), jnp.float32),
              jnp.full((1, QT), 3.0 * K, jnp.float32),
              ginv(hi0),
              jnp.zeros((1, QT), jnp.float32),
              jnp.int32(0))

    def q_cond(st):
        _, _, _, _, _, done, it = st
        return jnp.logical_and(jnp.any(done < 0.5), it < 64)

    def q_body(st):
        lo, hi, clo, chi, tstar, done, it = st
        t_lo, t_hi = ginv(lo), ginv(hi)
        denom = jnp.maximum(chi - clo, 1.0)
        tcand = t_lo + (t_hi - t_lo) * ((float(K) + 0.5) - clo) / denom
        pcand = jnp.where(it == 0, g_of(tmean), g_of(tcand))
        pbis = lo + lax.shift_right_logical(hi - lo, 1)
        pa = jnp.clip(pcand, lo + 1, jnp.maximum(lo + 1, hi - 1))
        pb = jnp.clip(pbis, lo + 1, jnp.maximum(lo + 1, hi - 1))
        p1 = jnp.minimum(pa, pb)
        p2 = jnp.maximum(pa, pb)
        t1, t2 = ginv(p1), ginv(p2)
        dd = d_ref[...]
        c1 = jnp.sum(jnp.where(dd <= t1, 1.0, 0.0), axis=0, keepdims=True)
        c2 = jnp.sum(jnp.where(dd <= t2, 1.0, 0.0), axis=0, keepdims=True)
        notdone = done < 0.5
        hit1 = c1 == float(K)
        hit2 = jnp.logical_and(c2 == float(K), jnp.logical_not(hit1))
        hit = jnp.logical_and(jnp.logical_or(hit1, hit2), notdone)
        gapdone = jnp.logical_and((hi - lo) <= 1,
                                  jnp.logical_and(notdone, jnp.logical_not(hit)))
        tstar = jnp.where(jnp.logical_and(notdone, hit1), t1,
                jnp.where(jnp.logical_and(notdone, hit2), t2,
                jnp.where(gapdone, t_hi, tstar)))
        done = jnp.maximum(done, jnp.where(
            jnp.logical_or(hit, gapdone), 1.0, 0.0))
        upd = done < 0.5
        b2lt = c2 < float(K)
        b1lt = c1 < float(K)
        lo = jnp.where(jnp.logical_and(upd, b2lt), p2,
             jnp.where(jnp.logical_and(upd, b1lt), p1, lo))
        clo = jnp.where(jnp.logical_and(upd, b2lt), c2,
              jnp.where(jnp.logical_and(upd, b1lt), c1, clo))
        g1 = jnp.logical_not(b1lt)
        g2 = jnp.logical_not(b2lt)
        hi = jnp.where(jnp.logical_and(upd, g1), p1,
             jnp.where(jnp.logical_and(upd, g2), p2, hi))
        chi = jnp.where(jnp.logical_and(upd, g1), c1,
              jnp.where(jnp.logical_and(upd, g2), c2, chi))
        return lo, hi, clo, chi, tstar, done, it + 1

    _, hi_f, _, _, tstar_f, done_f, _ = lax.while_loop(q_cond, q_body, state0)
    t_v = jnp.where(done_f > 0.5, tstar_f, ginv(hi_f))
    MT = (d_ref[...] <= t_v).astype(jnp.bfloat16)
    f = h_ref[0]
    st = st_ref[0]
    H = jnp.concatenate(
        [f, f * st[0:1], f * st[1:2], f * st[2:3]],
        axis=0).astype(jnp.bfloat16)
    ST = jnp.dot(H, MT, preferred_element_type=jnp.float32) * (1.0 / K)

    fq = fq_ref[0, :, 0:CP]

    G = ST[0:CP]
    A = jnp.concatenate([
        ST[CP:2 * CP] - q[0:1] * G,
        ST[2 * CP:3 * CP] - q[1:2] * G,
        ST[3 * CP:4 * CP] - q[2:3] * G,
    ], axis=0)
    term = jnp.dot(w2_ref[...], A, preferred_element_type=jnp.float32) + b2_ref[...]
    skip = lax.dot_general(wskip_ref[...], fq, (((1,), (1,)), ((), ())),
                           preferred_element_type=jnp.float32)
    out_ref[0] = jnp.maximum(term + skip, 0.0)


def _sc_gather_rows(table, idx):
    from jax.experimental.pallas import tpu_sc as plsc
    info = plsc.get_sparse_core_info()
    nw = info.num_cores * info.num_subcores
    bq = idx.shape[0]
    bpw = bq // nw
    mesh = plsc.VectorSubcoreMesh(core_axis_name="c", subcore_axis_name="s")

    @functools.partial(
        pl.kernel, mesh=mesh,
        out_type=jax.ShapeDtypeStruct((bq, table.shape[1]), jnp.float32),
        scratch_types=[
            pltpu.VMEM((bpw,), jnp.int32),
            pltpu.VMEM((bpw, table.shape[1]), jnp.float32),
            pltpu.SemaphoreType.DMA,
        ],
    )
    def gk(tab_hbm, idx_hbm, out_hbm, idx_v, rows_v, sem):
        wid = lax.axis_index("s") * info.num_cores + lax.axis_index("c")
        base = wid * bpw
        pltpu.sync_copy(idx_hbm.at[pl.ds(base, bpw)], idx_v)
        pltpu.async_copy(tab_hbm.at[idx_v], rows_v, sem).wait()
        pltpu.sync_copy(rows_v, out_hbm.at[pl.ds(base, bpw)])

    return gk(table, idx)


def kernel(query_xyz, support_xyz, features, query_idx, W0, b0, W1, b1, W2, b2, Wskip):
    B, NP, _ = query_xyz.shape
    N = support_xyz.shape[1]
    Cin = features.shape[1]
    Cmid = W1.shape[0]
    Cout = W2.shape[0]

    W1p = jnp.pad(W1, ((0, CP - Cmid), (0, 0)))
    b1p = jnp.pad(b1, (0, CP - Cmid))[:, None]
    W2p = jnp.pad(W2.reshape(Cout, 3, Cmid), ((0, 0), (0, 0), (0, CP - Cmid))
                  ).reshape(Cout, 3 * CP)
    Wskipp = jnp.pad(Wskip, ((0, 0), (0, CP - Cmid)))
    b0c = b0[:, None]
    b2c = b2[:, None]
    sT = jnp.transpose(support_xyz, (0, 2, 1))
    qT = jnp.transpose(query_xyz, (0, 2, 1))

    NT = 512
    H, fT = pl.pallas_call(
        _preconv_body,
        grid=(B, N // NT),
        in_specs=[
            pl.BlockSpec((1, Cin, NT), lambda b, n: (b, 0, n)),
            pl.BlockSpec((Cin, Cin), lambda b, n: (0, 0)),
            pl.BlockSpec((Cin, 1), lambda b, n: (0, 0)),
            pl.BlockSpec((CP, Cin), lambda b, n: (0, 0)),
            pl.BlockSpec((CP, 1), lambda b, n: (0, 0)),
        ],
        out_specs=[
            pl.BlockSpec((1, CP, NT), lambda b, n: (b, 0, n)),
            pl.BlockSpec((1, NT, 128), lambda b, n: (b, n, 0)),
        ],
        out_shape=[
            jax.ShapeDtypeStruct((B, CP, N), jnp.float32),
            jax.ShapeDtypeStruct((B, N, 128), jnp.float32),
        ],
        compiler_params=pltpu.CompilerParams(
            dimension_semantics=("parallel", "parallel")),
    )(features, W0, b0c, W1p, b1p)

    flat_idx = (query_idx.astype(jnp.int32)
                + jnp.arange(B, dtype=jnp.int32)[:, None] * N).reshape(-1)
    fq = _sc_gather_rows(fT.reshape(B * N, 128), flat_idx).reshape(B, NP, 128)

    out = pl.pallas_call(
        _assa_body,
        grid=(B, NP // QT),
        in_specs=[
            pl.BlockSpec((1, N, 3), lambda b, t: (b, 0, 0)),
            pl.BlockSpec((1, 3, N), lambda b, t: (b, 0, 0)),
            pl.BlockSpec((1, 3, QT), lambda b, t: (b, 0, t)),
            pl.BlockSpec((1, CP, N), lambda b, t: (b, 0, 0)),
            pl.BlockSpec((1, QT, 128), lambda b, t: (b, t, 0)),
            pl.BlockSpec((Cout, 3 * CP), lambda b, t: (0, 0)),
            pl.BlockSpec((Cout, 1), lambda b, t: (0, 0)),
            pl.BlockSpec((Cout, CP), lambda b, t: (0, 0)),
        ],
        out_specs=pl.BlockSpec((1, Cout, QT), lambda b, t: (b, 0, t)),
        out_shape=jax.ShapeDtypeStruct((B, Cout, NP), jnp.float32),
        scratch_shapes=[pltpu.VMEM((N, QT), jnp.float32)],
        compiler_params=pltpu.CompilerParams(
            dimension_semantics=("parallel", "parallel")),
    )(support_xyz, sT, qT, H, fq, W2p, b2c, Wskipp)
    return out
